# baseline (speedup 1.0000x reference)
import functools

import jax
import jax.numpy as jnp
from jax import lax
from jax.experimental import pallas as pl
from jax.experimental.pallas import tpu as pltpu
from jax.experimental.pallas import tpu_sc as plsc

_VOCAB = 100000
_N_EMBD = 768
_BLOCK = 2048
_BATCH = 4

_NC = 2
_NS = 16
_NW = _NC * _NS
_P = _BLOCK // _NW
_Q = 8
_NQ = _P // _Q
_DEPTH = 4
_G = _N_EMBD // 16
_WAVE = 4


def _emb_body(idx_hbm, tok_hbm, wpe_hbm, out_hbm,
              idx_v, wpe_v, tok_v,
              sg0, sg1, sg2, sg3, ss0, ss1, ss2, ss3, si):
    wid = lax.axis_index("s") * _NC + lax.axis_index("c")
    pos_base = wid * _P
    sg = (sg0, sg1, sg2, sg3)
    ss = (ss0, ss1, ss2, ss3)

    idx_h = [pltpu.async_copy(idx_hbm.at[pl.ds(b * _BLOCK + pos_base, _P)],
                              idx_v.at[pl.ds(b * _P, _P)], si)
             for b in range(_BATCH)]
    for h in idx_h:
        h.wait()

    def issue_chunk(c):
        r = c % _DEPTH
        hs = [pltpu.async_copy(
                  tok_hbm.at[idx_v.at[pl.ds(b * _P + c * _Q, _Q)]],
                  tok_v.at[r * _BATCH + b], sg[r])
              for b in range(_BATCH)]
        hs.append(pltpu.async_copy(
            wpe_hbm.at[pl.ds(pos_base + c * _Q, _Q)], wpe_v.at[r], sg[r]))
        return hs

    in_h = {0: issue_chunk(0), 1: issue_chunk(1)}
    store_h = {}

    for c in range(_NQ):
        r = c % _DEPTH
        if c + 2 < _NQ:
            if c - 2 >= 0:
                for h in store_h.pop(c - 2):
                    h.wait()
            in_h[c + 2] = issue_chunk(c + 2)
        for h in in_h.pop(c):
            h.wait()

        @plsc.parallel_loop(0, _Q)
        def add_row(t, r=r):
            for w in range(_G // _WAVE):
                sls = [pl.ds((w * _WAVE + k) * 16, 16) for k in range(_WAVE)]
                wps = [wpe_v[r, t, sl] for sl in sls]
                toks = [[tok_v[r * _BATCH + b, t, sl] for sl in sls]
                        for b in range(_BATCH)]
                for b in range(_BATCH):
                    for k in range(_WAVE):
                        tok_v[r * _BATCH + b, t, sls[k]] = toks[b][k] + wps[k]

        store_h[c] = [pltpu.async_copy(
                          tok_v.at[r * _BATCH + b],
                          out_hbm.at[pl.ds(b * _BLOCK + pos_base + c * _Q, _Q)],
                          ss[r])
                      for b in range(_BATCH)]

    for c in (_NQ - 2, _NQ - 1):
        for h in store_h.pop(c):
            h.wait()


_emb_call = functools.partial(
    pl.kernel,
    out_type=jax.ShapeDtypeStruct((_BATCH * _BLOCK, _N_EMBD), jnp.float32),
    mesh=plsc.VectorSubcoreMesh(core_axis_name="c", subcore_axis_name="s"),
    scratch_types=[
        pltpu.VMEM((_BATCH * _P,), jnp.int32),
        pltpu.VMEM((_DEPTH, _Q, _N_EMBD), jnp.float32),
        pltpu.VMEM((_DEPTH * _BATCH, _Q, _N_EMBD), jnp.float32),
        pltpu.SemaphoreType.DMA,
        pltpu.SemaphoreType.DMA,
        pltpu.SemaphoreType.DMA,
        pltpu.SemaphoreType.DMA,
        pltpu.SemaphoreType.DMA,
        pltpu.SemaphoreType.DMA,
        pltpu.SemaphoreType.DMA,
        pltpu.SemaphoreType.DMA,
        pltpu.SemaphoreType.DMA,
    ],
)(_emb_body)


def kernel(idx, tok_emb, wpe):
    idx_flat = idx.reshape(-1)
    out = _emb_call(idx_flat, tok_emb, wpe)
    return out.reshape(_BATCH, _BLOCK, _N_EMBD)


# speedup vs baseline: 1.0047x; 1.0047x over previous
import functools

import jax
import jax.numpy as jnp
from jax import lax
from jax.experimental import pallas as pl
from jax.experimental.pallas import tpu as pltpu
from jax.experimental.pallas import tpu_sc as plsc

_VOCAB = 100000
_N_EMBD = 768
_BLOCK = 2048
_BATCH = 4

_NC = 2
_NS = 16
_NW = _NC * _NS
_P = _BLOCK // _NW
_Q = 8
_NQ = _P // _Q
_DEPTH = 3
_G = _N_EMBD // 16
_WAVE = 4


def _emb_body(idx_hbm, tok_hbm, wpe_hbm, out_hbm,
              idx_v, wpe_v, tok_v, s0, s1, s2, si):
    wid = lax.axis_index("s") * _NC + lax.axis_index("c")
    pos_base = wid * _P
    sem = (s0, s1, s2)

    idx_h = [pltpu.async_copy(idx_hbm.at[pl.ds(b * _BLOCK + pos_base, _P)],
                              idx_v.at[pl.ds(b * _P, _P)], si)
             for b in range(_BATCH)]
    wpe_h = pltpu.async_copy(wpe_hbm.at[pl.ds(pos_base, _P)], wpe_v, si)
    for h in idx_h:
        h.wait()

    def issue_gathers(c):
        r = c % _DEPTH
        return [pltpu.async_copy(
                    tok_hbm.at[idx_v.at[pl.ds(b * _P + c * _Q, _Q)]],
                    tok_v.at[r * _BATCH + b], sem[r])
                for b in range(_BATCH)]

    in_h = {0: issue_gathers(0)}
    store_h = {}

    for c in range(_NQ):
        r = c % _DEPTH
        if c + 1 < _NQ:
            if c - 2 >= 0:
                for h in store_h.pop(c - 2):
                    h.wait()
            in_h[c + 1] = issue_gathers(c + 1)
        for h in in_h.pop(c):
            h.wait()
        if c == 0:
            wpe_h.wait()

        @plsc.parallel_loop(0, _Q)
        def add_row(t, r=r, c=c):
            for w in range(_G // _WAVE):
                sls = [pl.ds((w * _WAVE + k) * 16, 16) for k in range(_WAVE)]
                wps = [wpe_v[c * _Q + t, sl] for sl in sls]
                toks = [[tok_v[r * _BATCH + b, t, sl] for sl in sls]
                        for b in range(_BATCH)]
                for b in range(_BATCH):
                    for k in range(_WAVE):
                        tok_v[r * _BATCH + b, t, sls[k]] = toks[b][k] + wps[k]

        store_h[c] = [pltpu.async_copy(
                          tok_v.at[r * _BATCH + b],
                          out_hbm.at[pl.ds(b * _BLOCK + pos_base + c * _Q, _Q)],
                          sem[r])
                      for b in range(_BATCH)]

    for c in (_NQ - 3, _NQ - 2, _NQ - 1):
        for h in store_h.pop(c):
            h.wait()


_emb_call = functools.partial(
    pl.kernel,
    out_type=jax.ShapeDtypeStruct((_BATCH * _BLOCK, _N_EMBD), jnp.float32),
    mesh=plsc.VectorSubcoreMesh(core_axis_name="c", subcore_axis_name="s"),
    scratch_types=[
        pltpu.VMEM((_BATCH * _P,), jnp.int32),
        pltpu.VMEM((_P, _N_EMBD), jnp.float32),
        pltpu.VMEM((_DEPTH * _BATCH, _Q, _N_EMBD), jnp.float32),
        pltpu.SemaphoreType.DMA,
        pltpu.SemaphoreType.DMA,
        pltpu.SemaphoreType.DMA,
        pltpu.SemaphoreType.DMA,
    ],
)(_emb_body)


def kernel(idx, tok_emb, wpe):
    idx_flat = idx.reshape(-1)
    out = _emb_call(idx_flat, tok_emb, wpe)
    return out.reshape(_BATCH, _BLOCK, _N_EMBD)


# speedup vs baseline: 1.0883x; 1.0832x over previous
import functools

import jax
import jax.numpy as jnp
from jax import lax
from jax.experimental import pallas as pl
from jax.experimental.pallas import tpu as pltpu
from jax.experimental.pallas import tpu_sc as plsc

_VOCAB = 100000
_N_EMBD = 768
_BLOCK = 2048
_BATCH = 4

_NC = 2
_NS = 16
_NW = _NC * _NS
_P = _BLOCK // _NW
_Q = 8
_NQ = _P // _Q
_DEPTH = 4
_G = _N_EMBD // 16
_WAVE = 4


def _emb_body(idx_hbm, tok_hbm, wpe_hbm, out_hbm, idx_v, wpe_v, tok_v, sem):
    wid = lax.axis_index("s") * _NC + lax.axis_index("c")
    pos_base = wid * _P
    sidx = sem.at[_DEPTH]

    def gather_chunk(c, slot):
        for b in range(_BATCH):
            pltpu.async_copy(
                tok_hbm.at[idx_v.at[pl.ds(b * _P + c * _Q, _Q)]],
                tok_v.at[slot, b], sem.at[slot])
        pltpu.async_copy(wpe_hbm.at[pl.ds(pos_base + c * _Q, _Q)],
                         wpe_v.at[slot], sem.at[slot])

    def store_chunk(c, slot):
        for b in range(_BATCH):
            pltpu.async_copy(
                tok_v.at[slot, b],
                out_hbm.at[pl.ds(b * _BLOCK + pos_base + c * _Q, _Q)],
                sem.at[slot])

    def wait_k(slot, k):
        for _ in range(k):
            pltpu.make_async_copy(out_hbm.at[pl.ds(0, _Q)], tok_v.at[0, 0],
                                  sem.at[slot]).wait()

    idx_h = [pltpu.async_copy(idx_hbm.at[pl.ds(b * _BLOCK + pos_base, _P)],
                              idx_v.at[pl.ds(b * _P, _P)], sidx)
             for b in range(_BATCH)]
    for h in idx_h:
        h.wait()
    gather_chunk(0, 0)
    gather_chunk(1, 1)

    @pl.loop(0, _NQ)
    def chunk_loop(c):
        slot = c % _DEPTH

        @pl.when(c + 2 < _NQ)
        def _():
            nslot = (c + 2) % _DEPTH

            @pl.when(c >= 2)
            def _():
                wait_k(nslot, _BATCH)
            gather_chunk(c + 2, nslot)

        wait_k(slot, _BATCH + 1)

        for s in range(_DEPTH):
            @pl.when(slot == s)
            def _(s=s):
                @plsc.parallel_loop(0, _Q)
                def add_row(t):
                    for w in range(_G // _WAVE):
                        sls = [pl.ds((w * _WAVE + k) * 16, 16)
                               for k in range(_WAVE)]
                        wps = [wpe_v[s, t, sl] for sl in sls]
                        toks = [[tok_v[s, b, t, sl] for sl in sls]
                                for b in range(_BATCH)]
                        for b in range(_BATCH):
                            for k in range(_WAVE):
                                tok_v[s, b, t, sls[k]] = toks[b][k] + wps[k]

        store_chunk(c, slot)

    for c in range(_NQ - _DEPTH, _NQ):
        wait_k(c % _DEPTH, _BATCH)


_emb_call = functools.partial(
    pl.kernel,
    out_type=jax.ShapeDtypeStruct((_BATCH * _BLOCK, _N_EMBD), jnp.float32),
    mesh=plsc.VectorSubcoreMesh(core_axis_name="c", subcore_axis_name="s"),
    scratch_types=[
        pltpu.VMEM((_BATCH * _P,), jnp.int32),
        pltpu.VMEM((_DEPTH, _Q, _N_EMBD), jnp.float32),
        pltpu.VMEM((_DEPTH, _BATCH, _Q, _N_EMBD), jnp.float32),
        pltpu.SemaphoreType.DMA((_DEPTH + 1,)),
    ],
)(_emb_body)


def kernel(idx, tok_emb, wpe):
    idx_flat = idx.reshape(-1)
    out = _emb_call(idx_flat, tok_emb, wpe)
    return out.reshape(_BATCH, _BLOCK, _N_EMBD)


# speedup vs baseline: 1.1723x; 1.0772x over previous
import functools

import jax
import jax.numpy as jnp
from jax import lax
from jax.experimental import pallas as pl
from jax.experimental.pallas import tpu as pltpu
from jax.experimental.pallas import tpu_sc as plsc

_VOCAB = 100000
_N_EMBD = 768
_BLOCK = 2048
_BATCH = 4

_NC = 2
_NS = 16
_NW = _NC * _NS
_P = _BLOCK // _NW
_Q = 8
_NQ = _P // _Q
_DEPTH = 3
_G = _N_EMBD // 16
_WAVE = 4


def _emb_body(idx_hbm, tok_hbm, wpe_hbm, out_hbm, idx_v, wpe_v, tok_v, sem):
    wid = lax.axis_index("s") * _NC + lax.axis_index("c")
    pos_base = wid * _P

    idx_h = [pltpu.async_copy(idx_hbm.at[pl.ds(b * _BLOCK + pos_base, _P)],
                              idx_v.at[pl.ds(b * _P, _P)], sem.at[_DEPTH])
             for b in range(_BATCH)]
    wpe_h = pltpu.async_copy(wpe_hbm.at[pl.ds(pos_base, _P)], wpe_v,
                             sem.at[_DEPTH])
    for h in idx_h:
        h.wait()
    wpe_h.wait()

    def gather_chunk(c, slot):
        for b in range(_BATCH):
            pltpu.async_copy(
                tok_hbm.at[idx_v.at[pl.ds(b * _P + c * _Q, _Q)]],
                tok_v.at[slot, b], sem.at[slot])

    def store_chunk(c, slot):
        for b in range(_BATCH):
            pltpu.async_copy(
                tok_v.at[slot, b],
                out_hbm.at[pl.ds(b * _BLOCK + pos_base + c * _Q, _Q)],
                sem.at[slot])

    def wait_k(slot, k):
        for _ in range(k):
            pltpu.make_async_copy(out_hbm.at[pl.ds(0, _Q)], tok_v.at[0, 0],
                                  sem.at[slot]).wait()

    gather_chunk(0, 0)

    @pl.loop(0, _NQ)
    def chunk_loop(c):
        slot = lax.rem(c, _DEPTH)
        nslot = lax.rem(c + 1, _DEPTH)

        @pl.when(c >= _DEPTH - 1)
        def _():
            wait_k(nslot, _BATCH)

        @pl.when(c + 1 < _NQ)
        def _():
            gather_chunk(c + 1, nslot)

        wait_k(slot, _BATCH)

        @plsc.parallel_loop(0, _Q)
        def add_row(t):
            pos = c * _Q + t
            for w in range(_G // _WAVE):
                sls = [pl.ds((w * _WAVE + k) * 16, 16) for k in range(_WAVE)]
                wps = [wpe_v[pos, sl] for sl in sls]
                toks = [[tok_v[slot, b, t, sl] for sl in sls]
                        for b in range(_BATCH)]
                for b in range(_BATCH):
                    for k in range(_WAVE):
                        tok_v[slot, b, t, sls[k]] = toks[b][k] + wps[k]

        store_chunk(c, slot)

    for c in (_NQ - 2, _NQ - 1):
        wait_k(c % _DEPTH, _BATCH)


_emb_call = functools.partial(
    pl.kernel,
    out_type=jax.ShapeDtypeStruct((_BATCH * _BLOCK, _N_EMBD), jnp.float32),
    mesh=plsc.VectorSubcoreMesh(core_axis_name="c", subcore_axis_name="s"),
    scratch_types=[
        pltpu.VMEM((_BATCH * _P,), jnp.int32),
        pltpu.VMEM((_P, _N_EMBD), jnp.float32),
        pltpu.VMEM((_DEPTH, _BATCH, _Q, _N_EMBD), jnp.float32),
        pltpu.SemaphoreType.DMA((_DEPTH + 1,)),
    ],
)(_emb_body)


def kernel(idx, tok_emb, wpe):
    idx_flat = idx.reshape(-1)
    out = _emb_call(idx_flat, tok_emb, wpe)
    return out.reshape(_BATCH, _BLOCK, _N_EMBD)


# speedup vs baseline: 1.1755x; 1.0027x over previous
import functools

import jax
import jax.numpy as jnp
from jax import lax
from jax.experimental import pallas as pl
from jax.experimental.pallas import tpu as pltpu
from jax.experimental.pallas import tpu_sc as plsc

_VOCAB = 100000
_N_EMBD = 768
_BLOCK = 2048
_BATCH = 4

_NC = 2
_NS = 16
_NW = _NC * _NS
_P = _BLOCK // _NW
_Q = 8
_NQ = _P // _Q
_DEPTH = 3
_G = _N_EMBD // 16
_WAVE = 4


def _emb_body(idx_hbm, tok_hbm, wpe_hbm, out_hbm, idx_v, wpe_v, tok_v, sem):
    wid = lax.axis_index("s") * _NC + lax.axis_index("c")
    pos_base = wid * _P

    idx_h = [pltpu.async_copy(idx_hbm.at[pl.ds(b * _BLOCK + pos_base, _P)],
                              idx_v.at[pl.ds(b * _P, _P)], sem.at[_DEPTH])
             for b in range(_BATCH)]
    wpe_h = pltpu.async_copy(wpe_hbm.at[pl.ds(pos_base, _P)], wpe_v,
                             sem.at[_DEPTH + 1])
    for h in idx_h:
        h.wait()

    def gather_chunk(c, slot):
        for b in range(_BATCH):
            pltpu.async_copy(
                tok_hbm.at[idx_v.at[pl.ds(b * _P + c * _Q, _Q)]],
                tok_v.at[slot, b], sem.at[slot])

    def store_chunk(c, slot):
        for b in range(_BATCH):
            pltpu.async_copy(
                tok_v.at[slot, b],
                out_hbm.at[pl.ds(b * _BLOCK + pos_base + c * _Q, _Q)],
                sem.at[slot])

    def wait_k(slot, k):
        for _ in range(k):
            pltpu.make_async_copy(out_hbm.at[pl.ds(0, _Q)], tok_v.at[0, 0],
                                  sem.at[slot]).wait()

    gather_chunk(0, 0)
    gather_chunk(1, 1)
    wpe_h.wait()

    @pl.loop(0, _NQ)
    def chunk_loop(c):
        slot = lax.rem(c, _DEPTH)
        nslot = lax.rem(c + 1, _DEPTH)

        @pl.when(c >= _DEPTH - 1)
        def _():
            wait_k(nslot, _BATCH)

        @pl.when((c >= 1) & (c + 1 < _NQ))
        def _():
            gather_chunk(c + 1, nslot)

        wait_k(slot, _BATCH)

        for s in range(_DEPTH):
            @pl.when(slot == s)
            def _(s=s):
                @plsc.parallel_loop(0, _Q)
                def add_row(t):
                    pos = c * _Q + t
                    for w in range(_G // _WAVE):
                        sls = [pl.ds((w * _WAVE + k) * 16, 16)
                               for k in range(_WAVE)]
                        wps = [wpe_v[pos, sl] for sl in sls]
                        toks = [[tok_v[s, b, t, sl] for sl in sls]
                                for b in range(_BATCH)]
                        for b in range(_BATCH):
                            for k in range(_WAVE):
                                tok_v[s, b, t, sls[k]] = toks[b][k] + wps[k]

        store_chunk(c, slot)

    for c in (_NQ - 2, _NQ - 1):
        wait_k(c % _DEPTH, _BATCH)


_emb_call = functools.partial(
    pl.kernel,
    out_type=jax.ShapeDtypeStruct((_BATCH * _BLOCK, _N_EMBD), jnp.float32),
    mesh=plsc.VectorSubcoreMesh(core_axis_name="c", subcore_axis_name="s"),
    scratch_types=[
        pltpu.VMEM((_BATCH * _P,), jnp.int32),
        pltpu.VMEM((_P, _N_EMBD), jnp.float32),
        pltpu.VMEM((_DEPTH, _BATCH, _Q, _N_EMBD), jnp.float32),
        pltpu.SemaphoreType.DMA((_DEPTH + 2,)),
    ],
)(_emb_body)


def kernel(idx, tok_emb, wpe):
    idx_flat = idx.reshape(-1)
    out = _emb_call(idx_flat, tok_emb, wpe)
    return out.reshape(_BATCH, _BLOCK, _N_EMBD)
